# Initial kernel scaffold; baseline (speedup 1.0000x reference)
import jax
import jax.numpy as jnp
from jax.experimental import pallas as pl


def kernel(sid1, sid2, len1, len2, emb, W1, b1, W2, b2):
    raise NotImplementedError("write your pallas kernel here")


# speedup vs baseline: 14.1512x; 14.1512x over previous
import functools
import jax
import jax.numpy as jnp
from jax import lax
from jax.experimental import pallas as pl
from jax.experimental.pallas import tpu as pltpu
from jax.experimental.pallas import tpu_sc as plsc

_NC, _NS = 2, 16
_NT = _NC * _NS
_D = 128
_SL = 50
_LANES = 16
_NB = 4


def _embed_bag_sums(emb, sid3d, lens, zblk, padflat,
                    total_segs, interpret=False):
    S = total_segs // _NT
    ROUNDS = S * _SL // _D
    TRASH0 = _NS * S
    mesh = plsc.VectorSubcoreMesh(core_axis_name="c", subcore_axis_name="s",
                                  num_cores=_NC, num_subcores=_NS)

    @functools.partial(
        pl.kernel,
        mesh=mesh,
        compiler_params=pltpu.CompilerParams(needs_layout_passes=False),
        out_type=jax.ShapeDtypeStruct((total_segs, _D), jnp.float32),
        scratch_types=[
            pltpu.VMEM((ROUNDS, _D), jnp.int32),
            pltpu.VMEM((ROUNDS * _D,), jnp.int32),
            pltpu.VMEM((_NB, _D), jnp.int32),
            pltpu.VMEM((_NB, _D), jnp.int32),
            pltpu.VMEM((S,), jnp.int32),
            pltpu.VMEM((S,), jnp.int32),
            pltpu.VMEM((_NB, _D, _D), jnp.float32),
            pltpu.VMEM_SHARED((TRASH0 + _D, _D), jnp.float32),
            [pltpu.SemaphoreType.DMA] * _NB,
            [pltpu.SemaphoreType.DMA] * _NB,
        ],
        interpret=interpret,
    )
    def bag(emb_h, sid_h, len_h, z_h, pad_h, out_h,
            sid_v, cpack_v, idx_r, dst_r, len_v, soff_v, rows_v, acc_s,
            gsems, ssems):
        c = lax.axis_index("c")
        s = lax.axis_index("s")
        t = c * _NS + s
        base_dst = s * S

        pltpu.sync_copy(sid_h.at[t], sid_v)
        pltpu.sync_copy(len_h.at[pl.ds(t * S, S)], len_v)
        pltpu.sync_copy(pad_h, cpack_v)

        for blk in range(S // _D):
            pltpu.sync_copy(z_h, acc_s.at[pl.ds(s * S + blk * _D, _D)])

        carry = jnp.int32(0)
        for i in range(S // _LANES):
            ln = len_v[pl.ds(i * _LANES, _LANES)]
            csum = jnp.cumsum(ln)
            soff_v[pl.ds(i * _LANES, _LANES)] = carry + csum - ln
            carry = carry + jnp.max(csum)
        n_valid = carry

        def comp(jr, st):
            seg0, tok0 = st
            for cc in range(_D // _LANES):
                w = sid_v[jr, pl.ds(cc * _LANES, _LANES)]
                tv = lax.iota(jnp.int32, _LANES) + (tok0 + cc * _LANES)
                adj = ((tv >= _SL).astype(jnp.int32)
                       + (tv >= 2 * _SL).astype(jnp.int32)
                       + (tv >= 3 * _SL).astype(jnp.int32))
                seg = seg0 + adj
                tok = tv - adj * _SL
                ln = plsc.load_gather(len_v, [seg])
                so = plsc.load_gather(soff_v, [seg])
                packed = (w << 14) | (seg << 6) | tok
                plsc.store_scatter(cpack_v, [so + tok], packed,
                                   mask=tok < ln)
            t2 = tok0 + (_D - 2 * _SL)
            wrap = (t2 >= _SL).astype(jnp.int32)
            return (seg0 + 2 + wrap, t2 - wrap * _SL)

        lax.fori_loop(0, ROUNDS, comp,
                      (jnp.int32(0), jnp.int32(0)))
        nr = (n_valid + (_D - 1)) >> 7
        nr = jnp.maximum((nr + (_NB - 1)) & (-_NB), _NB)

        def unpack_ring(r, slot):
            for cc in range(_D // _LANES):
                sl = pl.ds(cc * _LANES, _LANES)
                posv = lax.iota(jnp.int32, _LANES) + (r * _D + cc * _LANES)
                w = plsc.load_gather(cpack_v, [posv])
                tok = w & 63
                seg = (w >> 6) & 255
                trashv = lax.iota(jnp.int32, _LANES) + (TRASH0 + cc * _LANES)
                idx_r[slot, sl] = w >> 14
                dst_r[slot, sl] = jnp.where(tok == 63, trashv, seg + base_dst)

        for b in range(_NB - 1):
            unpack_ring(b, b)
            pltpu.async_copy(emb_h.at[idx_r.at[b]], rows_v.at[b], gsems[b])

        def wait_g(sem):
            pltpu.make_async_copy(emb_h.at[idx_r.at[0]], rows_v.at[0],
                                  sem).wait()

        def wait_s(sem):
            pltpu.make_async_copy(rows_v.at[0], acc_s.at[pl.ds(0, _D)],
                                  sem).wait()

        def ring_fn(k, carry2):
            for p in range(_NB):
                r = _NB * k + p
                bn = (p + _NB - 1) % _NB
                wait_g(gsems[p])
                pltpu.async_copy(rows_v.at[p], acc_s.at[dst_r.at[p]],
                                 ssems[p], add=True)

                @pl.when(r + _NB - 1 < nr)
                def _():
                    @pl.when(r > 0)
                    def _():
                        wait_s(ssems[bn])
                    unpack_ring(r + _NB - 1, bn)
                    pltpu.async_copy(emb_h.at[idx_r.at[bn]],
                                     rows_v.at[bn], gsems[bn])
            return carry2

        lax.fori_loop(0, nr >> (_NB.bit_length() - 1), ring_fn, 0)
        for b in range(_NB):
            wait_s(ssems[b])

        pltpu.sync_copy(acc_s.at[pl.ds(s * S, S)], out_h.at[pl.ds(t * S, S)])

    return bag(emb, sid3d, lens, zblk, padflat)


def _dense(sums, len1c, len2c, W1, b1, W2, b2, interpret=False):
    Bn2, D = sums.shape
    Bn = Bn2 // 2
    H = W1.shape[1]
    C = W2.shape[1]
    BLK = 256
    nblk = Bn // BLK

    def body(us_r, vs_r, l1_r, l2_r, W1_r, b1_r, W2_r, b2_r, y_r, u_r, v_r):
        d1 = jnp.maximum(l1_r[...], 1).astype(jnp.float32)
        d2 = jnp.maximum(l2_r[...], 1).astype(jnp.float32)
        u = us_r[...] / d1
        v = vs_r[...] / d2
        u_r[...] = u
        v_r[...] = v
        a = jnp.abs(u - v)
        m = u * v
        h = (jnp.dot(u, W1_r[0:D, :], preferred_element_type=jnp.float32)
             + jnp.dot(v, W1_r[D:2 * D, :], preferred_element_type=jnp.float32)
             + jnp.dot(a, W1_r[2 * D:3 * D, :], preferred_element_type=jnp.float32)
             + jnp.dot(m, W1_r[3 * D:4 * D, :], preferred_element_type=jnp.float32)
             + b1_r[...])
        h = jnp.maximum(h, 0.0)
        y_r[...] = jnp.dot(h, W2_r[...], preferred_element_type=jnp.float32) + b2_r[...]

    return pl.pallas_call(
        body,
        grid=(nblk,),
        in_specs=[
            pl.BlockSpec((BLK, D), lambda i: (i, 0)),
            pl.BlockSpec((BLK, D), lambda i: (i + nblk, 0)),
            pl.BlockSpec((BLK, 1), lambda i: (i, 0)),
            pl.BlockSpec((BLK, 1), lambda i: (i, 0)),
            pl.BlockSpec((4 * D, H), lambda i: (0, 0)),
            pl.BlockSpec((1, H), lambda i: (0, 0)),
            pl.BlockSpec((H, C), lambda i: (0, 0)),
            pl.BlockSpec((1, C), lambda i: (0, 0)),
        ],
        out_specs=[
            pl.BlockSpec((BLK, C), lambda i: (i, 0)),
            pl.BlockSpec((BLK, D), lambda i: (i, 0)),
            pl.BlockSpec((BLK, D), lambda i: (i, 0)),
        ],
        out_shape=[
            jax.ShapeDtypeStruct((Bn, C), jnp.float32),
            jax.ShapeDtypeStruct((Bn, D), jnp.float32),
            jax.ShapeDtypeStruct((Bn, D), jnp.float32),
        ],
        interpret=interpret,
    )(sums, sums, len1c, len2c, W1, b1, W2, b2)


def kernel(sid1, sid2, len1, len2, emb, W1, b1, W2, b2, interpret=False):
    B, L = sid1.shape
    rounds = (2 * B // _NT) * _SL // _D
    sid3d = jnp.concatenate([sid1, sid2], axis=0).astype(jnp.int32)
    sid3d = sid3d.reshape(_NT, rounds, _D)
    lens = jnp.concatenate([len1, len2], axis=0).astype(jnp.int32)
    l1 = len1.astype(jnp.int32)
    l2 = len2.astype(jnp.int32)
    zblk = jnp.zeros((_D, _D), jnp.float32)
    padflat = ((jnp.arange(rounds * _D, dtype=jnp.int32) & 8191) << 14) | 63

    sums = _embed_bag_sums(emb, sid3d, lens, zblk, padflat,
                           2 * B, interpret=interpret)
    y, u, v = _dense(sums, l1[:, None], l2[:, None], W1, b1[None, :],
                     W2, b2[None, :], interpret=interpret)
    return (y, u, v)


# speedup vs baseline: 14.5999x; 1.0317x over previous
import functools
import jax
import jax.numpy as jnp
from jax import lax
from jax.experimental import pallas as pl
from jax.experimental.pallas import tpu as pltpu
from jax.experimental.pallas import tpu_sc as plsc

_NC, _NS = 2, 16
_NT = _NC * _NS
_D = 128
_SL = 50
_LANES = 16
_NB = 5


def _embed_bag_sums(emb, sid3d, lens, zblk, total_segs, interpret=False):
    S = total_segs // _NT
    ROUNDS = S * _SL // _D
    TRASH0 = _NS * S
    mesh = plsc.VectorSubcoreMesh(core_axis_name="c", subcore_axis_name="s",
                                  num_cores=_NC, num_subcores=_NS)

    @functools.partial(
        pl.kernel,
        mesh=mesh,
        compiler_params=pltpu.CompilerParams(needs_layout_passes=False),
        out_type=jax.ShapeDtypeStruct((total_segs, _D), jnp.float32),
        scratch_types=[
            pltpu.VMEM((ROUNDS, _D), jnp.int32),
            pltpu.VMEM((_NB, _D), jnp.int32),
            pltpu.VMEM((_NB, _D), jnp.int32),
            pltpu.VMEM((S,), jnp.int32),
            pltpu.VMEM((S,), jnp.int32),
            pltpu.VMEM((_NB, _D, _D), jnp.float32),
            pltpu.VMEM_SHARED((TRASH0 + 2 * _LANES, _D), jnp.float32),
            [pltpu.SemaphoreType.DMA] * _NB,
            [pltpu.SemaphoreType.DMA] * _NB,
        ],
        interpret=interpret,
    )
    def bag(emb_h, sid_h, len_h, z_h, out_h,
            cpack_v, idx_r, dst_r, len_v, soff_v, rows_v, acc_s,
            gsems, ssems):
        c = lax.axis_index("c")
        s = lax.axis_index("s")
        t = c * _NS + s
        base_dst = s * S

        pltpu.sync_copy(sid_h.at[t], cpack_v)
        pltpu.sync_copy(len_h.at[pl.ds(t * S, S)], len_v)

        for blk in range(S // _D):
            pltpu.sync_copy(z_h, acc_s.at[pl.ds(s * S + blk * _D, _D)])

        carry = jnp.int32(0)
        for i in range(S // _LANES):
            ln = len_v[pl.ds(i * _LANES, _LANES)]
            csum = jnp.cumsum(ln)
            soff_v[pl.ds(i * _LANES, _LANES)] = carry + csum - ln
            carry = carry + jnp.max(csum)
        n_valid = carry

        def comp(jr, st):
            seg0, tok0 = st
            for cc in range(_D // _LANES):
                w = cpack_v[jr, pl.ds(cc * _LANES, _LANES)]
                tv = lax.iota(jnp.int32, _LANES) + (tok0 + cc * _LANES)
                adj = ((tv >= _SL).astype(jnp.int32)
                       + (tv >= 2 * _SL).astype(jnp.int32)
                       + (tv >= 3 * _SL).astype(jnp.int32))
                seg = seg0 + adj
                tok = tv - adj * _SL
                ln = plsc.load_gather(len_v, [seg])
                so = plsc.load_gather(soff_v, [seg])
                packed = (w << 14) | (seg << 6) | tok
                pos = so + tok
                plsc.store_scatter(cpack_v, [pos >> 7, pos & (_D - 1)],
                                   packed, mask=tok < ln)
            t2 = tok0 + (_D - 2 * _SL)
            wrap = (t2 >= _SL).astype(jnp.int32)
            return (seg0 + 2 + wrap, t2 - wrap * _SL)

        lax.fori_loop(0, ROUNDS, comp,
                      (jnp.int32(0), jnp.int32(0)))

        for i in range(_NB * _D // _LANES):
            posv = n_valid + (lax.iota(jnp.int32, _LANES) + i * _LANES)
            padv = ((posv & 8191) << 14) | 63
            plsc.store_scatter(cpack_v, [posv >> 7, posv & (_D - 1)], padv,
                               mask=posv < ROUNDS * _D)
        nr0 = (n_valid + (_D - 1)) >> 7
        kk = jnp.maximum(((nr0 + (_NB - 1)) * 52429) >> 18, 1)
        nr = kk * _NB

        def unpack_ring(r, slot):
            for cc in range(_D // _LANES):
                sl = pl.ds(cc * _LANES, _LANES)
                posv = lax.iota(jnp.int32, _LANES) + (r * _D + cc * _LANES)
                w = plsc.load_gather(cpack_v,
                                     [posv >> 7, posv & (_D - 1)])
                tok = w & 63
                seg = (w >> 6) & 255
                trashv = lax.iota(jnp.int32, _LANES) + (TRASH0
                                                        + (cc % 2) * _LANES)
                idx_r[slot, sl] = w >> 14
                dst_r[slot, sl] = jnp.where(tok == 63, trashv, seg + base_dst)

        for b in range(_NB - 1):
            unpack_ring(b, b)
            pltpu.async_copy(emb_h.at[idx_r.at[b]], rows_v.at[b], gsems[b])

        def wait_g(sem):
            pltpu.make_async_copy(emb_h.at[idx_r.at[0]], rows_v.at[0],
                                  sem).wait()

        def wait_s(sem):
            pltpu.make_async_copy(rows_v.at[0], acc_s.at[pl.ds(0, _D)],
                                  sem).wait()

        def ring_fn(k, carry2):
            for p in range(_NB):
                r = _NB * k + p
                bn = (p + _NB - 1) % _NB
                wait_g(gsems[p])
                pltpu.async_copy(rows_v.at[p], acc_s.at[dst_r.at[p]],
                                 ssems[p], add=True)

                @pl.when(r + _NB - 1 < nr)
                def _():
                    @pl.when(r > 0)
                    def _():
                        wait_s(ssems[bn])
                    unpack_ring(r + _NB - 1, bn)
                    pltpu.async_copy(emb_h.at[idx_r.at[bn]],
                                     rows_v.at[bn], gsems[bn])
            return carry2

        lax.fori_loop(0, kk, ring_fn, 0)
        for b in range(_NB):
            wait_s(ssems[b])

        pltpu.sync_copy(acc_s.at[pl.ds(s * S, S)], out_h.at[pl.ds(t * S, S)])

    return bag(emb, sid3d, lens, zblk)


def _dense(sums, len1c, len2c, W1, b1, W2, b2, interpret=False):
    Bn2, D = sums.shape
    Bn = Bn2 // 2
    H = W1.shape[1]
    C = W2.shape[1]
    BLK = 256
    nblk = Bn // BLK

    def body(us_r, vs_r, l1_r, l2_r, W1_r, b1_r, W2_r, b2_r, y_r, u_r, v_r):
        d1 = jnp.maximum(l1_r[...], 1).astype(jnp.float32)
        d2 = jnp.maximum(l2_r[...], 1).astype(jnp.float32)
        u = us_r[...] / d1
        v = vs_r[...] / d2
        u_r[...] = u
        v_r[...] = v
        a = jnp.abs(u - v)
        m = u * v
        h = (jnp.dot(u, W1_r[0:D, :], preferred_element_type=jnp.float32)
             + jnp.dot(v, W1_r[D:2 * D, :], preferred_element_type=jnp.float32)
             + jnp.dot(a, W1_r[2 * D:3 * D, :], preferred_element_type=jnp.float32)
             + jnp.dot(m, W1_r[3 * D:4 * D, :], preferred_element_type=jnp.float32)
             + b1_r[...])
        h = jnp.maximum(h, 0.0)
        y_r[...] = jnp.dot(h, W2_r[...], preferred_element_type=jnp.float32) + b2_r[...]

    return pl.pallas_call(
        body,
        grid=(nblk,),
        in_specs=[
            pl.BlockSpec((BLK, D), lambda i: (i, 0)),
            pl.BlockSpec((BLK, D), lambda i: (i + nblk, 0)),
            pl.BlockSpec((BLK, 1), lambda i: (i, 0)),
            pl.BlockSpec((BLK, 1), lambda i: (i, 0)),
            pl.BlockSpec((4 * D, H), lambda i: (0, 0)),
            pl.BlockSpec((1, H), lambda i: (0, 0)),
            pl.BlockSpec((H, C), lambda i: (0, 0)),
            pl.BlockSpec((1, C), lambda i: (0, 0)),
        ],
        out_specs=[
            pl.BlockSpec((BLK, C), lambda i: (i, 0)),
            pl.BlockSpec((BLK, D), lambda i: (i, 0)),
            pl.BlockSpec((BLK, D), lambda i: (i, 0)),
        ],
        out_shape=[
            jax.ShapeDtypeStruct((Bn, C), jnp.float32),
            jax.ShapeDtypeStruct((Bn, D), jnp.float32),
            jax.ShapeDtypeStruct((Bn, D), jnp.float32),
        ],
        interpret=interpret,
    )(sums, sums, len1c, len2c, W1, b1, W2, b2)


def kernel(sid1, sid2, len1, len2, emb, W1, b1, W2, b2, interpret=False):
    B, L = sid1.shape
    rounds = (2 * B // _NT) * _SL // _D
    sid3d = jnp.concatenate([sid1, sid2], axis=0).astype(jnp.int32)
    sid3d = sid3d.reshape(_NT, rounds, _D)
    lens = jnp.concatenate([len1, len2], axis=0).astype(jnp.int32)
    l1 = len1.astype(jnp.int32)
    l2 = len2.astype(jnp.int32)
    zblk = jnp.zeros((_D, _D), jnp.float32)

    sums = _embed_bag_sums(emb, sid3d, lens, zblk, 2 * B,
                           interpret=interpret)
    y, u, v = _dense(sums, l1[:, None], l2[:, None], W1, b1[None, :],
                     W2, b2[None, :], interpret=interpret)
    return (y, u, v)


# speedup vs baseline: 15.0835x; 1.0331x over previous
import functools
import jax
import jax.numpy as jnp
from jax import lax
from jax.experimental import pallas as pl
from jax.experimental.pallas import tpu as pltpu
from jax.experimental.pallas import tpu_sc as plsc

_NC, _NS = 2, 16
_NT = _NC * _NS
_D = 128
_SL = 50
_LANES = 16
_NB = 5


def _embed_bag_sums(emb, sid3d, lens, zblk, total_segs, interpret=False):
    S = total_segs // _NT
    ROUNDS = S * _SL // _D
    TRASH0 = _NS * S
    mesh = plsc.VectorSubcoreMesh(core_axis_name="c", subcore_axis_name="s",
                                  num_cores=_NC, num_subcores=_NS)

    @functools.partial(
        pl.kernel,
        mesh=mesh,
        compiler_params=pltpu.CompilerParams(needs_layout_passes=False),
        out_type=jax.ShapeDtypeStruct((total_segs, _D), jnp.float32),
        scratch_types=[
            pltpu.VMEM((ROUNDS, _D), jnp.int32),
            pltpu.VMEM((_NB, _D), jnp.int32),
            pltpu.VMEM((_NB, _D), jnp.int32),
            pltpu.VMEM((S,), jnp.int32),
            pltpu.VMEM((S,), jnp.int32),
            pltpu.VMEM((_NB, _D, _D), jnp.float32),
            pltpu.VMEM_SHARED((TRASH0 + 2 * _LANES, _D), jnp.float32),
            [pltpu.SemaphoreType.DMA] * _NB,
            [pltpu.SemaphoreType.DMA] * _NB,
        ],
        interpret=interpret,
    )
    def bag(emb_h, sid_h, len_h, z_h, out_h,
            cpack_v, idx_r, dst_r, len_v, soff_v, rows_v, acc_s,
            gsems, ssems):
        c = lax.axis_index("c")
        s = lax.axis_index("s")
        t = c * _NS + s
        base_dst = s * S

        pltpu.sync_copy(sid_h.at[t], cpack_v)
        pltpu.sync_copy(len_h.at[pl.ds(t * S, S)], len_v)

        for blk in range(S // _D):
            pltpu.sync_copy(z_h, acc_s.at[pl.ds(s * S + blk * _D, _D)])

        carry = jnp.int32(0)
        for i in range(S // _LANES):
            ln = len_v[pl.ds(i * _LANES, _LANES)]
            csum = jnp.cumsum(ln)
            soff_v[pl.ds(i * _LANES, _LANES)] = carry + csum - ln
            carry = carry + jnp.max(csum)
        n_valid = carry

        def comp(jr, st):
            seg0, tok0 = st
            for cc in range(_D // _LANES):
                w = cpack_v[jr, pl.ds(cc * _LANES, _LANES)]
                tv = lax.iota(jnp.int32, _LANES) + (tok0 + cc * _LANES)
                adj = ((tv >= _SL).astype(jnp.int32)
                       + (tv >= 2 * _SL).astype(jnp.int32)
                       + (tv >= 3 * _SL).astype(jnp.int32))
                seg = seg0 + adj
                tok = tv - adj * _SL
                ln = plsc.load_gather(len_v, [seg])
                so = plsc.load_gather(soff_v, [seg])
                packed = (w << 14) | (seg << 6) | tok
                pos = so + tok
                plsc.store_scatter(cpack_v, [pos >> 7, pos & (_D - 1)],
                                   packed, mask=tok < ln)
            t2 = tok0 + (_D - 2 * _SL)
            wrap = (t2 >= _SL).astype(jnp.int32)
            return (seg0 + 2 + wrap, t2 - wrap * _SL)

        lax.fori_loop(0, ROUNDS, comp,
                      (jnp.int32(0), jnp.int32(0)))

        for i in range(_NB * _D // _LANES):
            posv = n_valid + (lax.iota(jnp.int32, _LANES) + i * _LANES)
            padv = ((posv & 8191) << 14) | 63
            plsc.store_scatter(cpack_v, [posv >> 7, posv & (_D - 1)], padv,
                               mask=posv < ROUNDS * _D)
        nr0 = (n_valid + (_D - 1)) >> 7
        kk = jnp.maximum(((nr0 + (_NB - 1)) * 52429) >> 18, 1)
        nr = kk * _NB

        def unpack_ring(r, slot):
            for cc in range(_D // _LANES):
                sl = pl.ds(cc * _LANES, _LANES)
                posv = lax.iota(jnp.int32, _LANES) + (r * _D + cc * _LANES)
                w = plsc.load_gather(cpack_v,
                                     [posv >> 7, posv & (_D - 1)])
                tok = w & 63
                seg = (w >> 6) & 255
                trashv = lax.iota(jnp.int32, _LANES) + (TRASH0
                                                        + (cc % 2) * _LANES)
                idx_r[slot, sl] = w >> 14
                dst_r[slot, sl] = jnp.where(tok == 63, trashv, seg + base_dst)

        for b in range(_NB - 1):
            unpack_ring(b, b)
            pltpu.async_copy(emb_h.at[idx_r.at[b]], rows_v.at[b], gsems[b])

        def wait_g(sem):
            pltpu.make_async_copy(emb_h.at[idx_r.at[0]], rows_v.at[0],
                                  sem).wait()

        def wait_s(sem):
            pltpu.make_async_copy(rows_v.at[0], acc_s.at[pl.ds(0, _D)],
                                  sem).wait()

        def ring_fn(k, carry2):
            for p in range(_NB):
                r = _NB * k + p
                bn = (p + _NB - 1) % _NB
                wait_g(gsems[p])
                pltpu.async_copy(rows_v.at[p], acc_s.at[dst_r.at[p]],
                                 ssems[p], add=True)

                @pl.when(r + _NB - 1 < nr)
                def _():
                    @pl.when(r > 0)
                    def _():
                        wait_s(ssems[bn])
                    unpack_ring(r + _NB - 1, bn)
                    pltpu.async_copy(emb_h.at[idx_r.at[bn]],
                                     rows_v.at[bn], gsems[bn])
            return carry2

        lax.fori_loop(0, kk, ring_fn, 0)
        for b in range(_NB):
            wait_s(ssems[b])

        pltpu.sync_copy(acc_s.at[pl.ds(s * S, S)], out_h.at[pl.ds(t * S, S)])

    return bag(emb, sid3d, lens, zblk)


def _dense(sums, len1c, len2c, W1, b1, W2, b2, interpret=False):
    Bn2, D = sums.shape
    Bn = Bn2 // 2
    H = W1.shape[1]
    C = W2.shape[1]
    BLK = 512
    nblk = Bn // BLK

    def body(us_r, vs_r, l1_r, l2_r, W1_r, b1_r, W2_r, b2_r, y_r, u_r, v_r):
        d1 = jnp.maximum(l1_r[...], 1).astype(jnp.float32)
        d2 = jnp.maximum(l2_r[...], 1).astype(jnp.float32)
        u = us_r[...] / d1
        v = vs_r[...] / d2
        u_r[...] = u
        v_r[...] = v
        a = jnp.abs(u - v)
        m = u * v
        h = (jnp.dot(u, W1_r[0:D, :], preferred_element_type=jnp.float32)
             + jnp.dot(v, W1_r[D:2 * D, :], preferred_element_type=jnp.float32)
             + jnp.dot(a, W1_r[2 * D:3 * D, :], preferred_element_type=jnp.float32)
             + jnp.dot(m, W1_r[3 * D:4 * D, :], preferred_element_type=jnp.float32)
             + b1_r[...])
        h = jnp.maximum(h, 0.0)
        y_r[...] = jnp.dot(h, W2_r[...], preferred_element_type=jnp.float32) + b2_r[...]

    return pl.pallas_call(
        body,
        grid=(nblk,),
        in_specs=[
            pl.BlockSpec((BLK, D), lambda i: (i, 0)),
            pl.BlockSpec((BLK, D), lambda i: (i + nblk, 0)),
            pl.BlockSpec((BLK, 1), lambda i: (i, 0)),
            pl.BlockSpec((BLK, 1), lambda i: (i, 0)),
            pl.BlockSpec((4 * D, H), lambda i: (0, 0)),
            pl.BlockSpec((1, H), lambda i: (0, 0)),
            pl.BlockSpec((H, C), lambda i: (0, 0)),
            pl.BlockSpec((1, C), lambda i: (0, 0)),
        ],
        out_specs=[
            pl.BlockSpec((BLK, C), lambda i: (i, 0)),
            pl.BlockSpec((BLK, D), lambda i: (i, 0)),
            pl.BlockSpec((BLK, D), lambda i: (i, 0)),
        ],
        out_shape=[
            jax.ShapeDtypeStruct((Bn, C), jnp.float32),
            jax.ShapeDtypeStruct((Bn, D), jnp.float32),
            jax.ShapeDtypeStruct((Bn, D), jnp.float32),
        ],
        interpret=interpret,
    )(sums, sums, len1c, len2c, W1, b1, W2, b2)


def kernel(sid1, sid2, len1, len2, emb, W1, b1, W2, b2, interpret=False):
    B, L = sid1.shape
    rounds = (2 * B // _NT) * _SL // _D
    sid3d = jnp.concatenate([sid1, sid2], axis=0).astype(jnp.int32)
    sid3d = sid3d.reshape(_NT, rounds, _D)
    lens = jnp.concatenate([len1, len2], axis=0).astype(jnp.int32)
    l1 = len1.astype(jnp.int32)
    l2 = len2.astype(jnp.int32)
    zblk = jnp.zeros((_D, _D), jnp.float32)

    sums = _embed_bag_sums(emb, sid3d, lens, zblk, 2 * B,
                           interpret=interpret)
    y, u, v = _dense(sums, l1[:, None], l2[:, None], W1, b1[None, :],
                     W2, b2[None, :], interpret=interpret)
    return (y, u, v)


# speedup vs baseline: 15.3681x; 1.0189x over previous
import functools
import jax
import jax.numpy as jnp
from jax import lax
from jax.experimental import pallas as pl
from jax.experimental.pallas import tpu as pltpu
from jax.experimental.pallas import tpu_sc as plsc

_NC, _NS = 2, 16
_NT = _NC * _NS
_D = 128
_SL = 50
_LANES = 16
_NB = 5


def _embed_bag_sums(emb, sid3d, lens, zblk, total_segs, interpret=False):
    S = total_segs // _NT
    ROUNDS = S * _SL // _D
    TRASH0 = _NS * S
    mesh = plsc.VectorSubcoreMesh(core_axis_name="c", subcore_axis_name="s",
                                  num_cores=_NC, num_subcores=_NS)

    @functools.partial(
        pl.kernel,
        mesh=mesh,
        compiler_params=pltpu.CompilerParams(needs_layout_passes=False),
        out_type=jax.ShapeDtypeStruct((total_segs, _D), jnp.float32),
        scratch_types=[
            pltpu.VMEM((ROUNDS, _D), jnp.int32),
            pltpu.VMEM((_NB, _D), jnp.int32),
            pltpu.VMEM((_NB, _D), jnp.int32),
            pltpu.VMEM((S,), jnp.int32),
            pltpu.VMEM((S,), jnp.int32),
            pltpu.VMEM((_NB, _D, _D), jnp.float32),
            pltpu.VMEM_SHARED((TRASH0 + 2 * _LANES, _D), jnp.float32),
            [pltpu.SemaphoreType.DMA] * _NB,
            [pltpu.SemaphoreType.DMA] * _NB,
        ],
        interpret=interpret,
    )
    def bag(emb_h, sid_h, len_h, z_h, out_h,
            cpack_v, idx_r, dst_r, len_v, soff_v, rows_v, acc_s,
            gsems, ssems):
        c = lax.axis_index("c")
        s = lax.axis_index("s")
        t = c * _NS + s
        base_dst = s * S

        pltpu.sync_copy(sid_h.at[t], cpack_v)
        pltpu.sync_copy(len_h.at[pl.ds(t * S, S)], len_v)

        for blk in range(S // _D):
            pltpu.sync_copy(z_h, acc_s.at[pl.ds(s * S + blk * _D, _D)])

        carry = jnp.int32(0)
        for i in range(S // _LANES):
            ln = len_v[pl.ds(i * _LANES, _LANES)]
            csum = jnp.cumsum(ln)
            soff_v[pl.ds(i * _LANES, _LANES)] = carry + csum - ln
            carry = carry + jnp.max(csum)
        n_valid = carry

        def comp(jr, st):
            seg0, tok0 = st
            for cc in range(_D // _LANES):
                w = cpack_v[jr, pl.ds(cc * _LANES, _LANES)]
                tv = lax.iota(jnp.int32, _LANES) + (tok0 + cc * _LANES)
                adj = ((tv >= _SL).astype(jnp.int32)
                       + (tv >= 2 * _SL).astype(jnp.int32)
                       + (tv >= 3 * _SL).astype(jnp.int32))
                seg = seg0 + adj
                tok = tv - adj * _SL
                ln = plsc.load_gather(len_v, [seg])
                so = plsc.load_gather(soff_v, [seg])
                packed = (w << 14) | (seg << 6) | tok
                pos = so + tok
                plsc.store_scatter(cpack_v, [pos >> 7, pos & (_D - 1)],
                                   packed, mask=tok < ln)
            t2 = tok0 + (_D - 2 * _SL)
            wrap = (t2 >= _SL).astype(jnp.int32)
            return (seg0 + 2 + wrap, t2 - wrap * _SL)

        lax.fori_loop(0, ROUNDS, comp,
                      (jnp.int32(0), jnp.int32(0)))

        for i in range(_NB * _D // _LANES):
            posv = n_valid + (lax.iota(jnp.int32, _LANES) + i * _LANES)
            padv = ((posv & 8191) << 14) | 63
            plsc.store_scatter(cpack_v, [posv >> 7, posv & (_D - 1)], padv,
                               mask=posv < ROUNDS * _D)
        nr0 = (n_valid + (_D - 1)) >> 7
        kk = jnp.maximum(((nr0 + (_NB - 1)) * 52429) >> 18, 1)
        nr = kk * _NB

        def unpack_ring(r, slot):
            for cc in range(_D // _LANES):
                sl = pl.ds(cc * _LANES, _LANES)
                posv = lax.iota(jnp.int32, _LANES) + (r * _D + cc * _LANES)
                w = plsc.load_gather(cpack_v,
                                     [posv >> 7, posv & (_D - 1)])
                tok = w & 63
                seg = (w >> 6) & 255
                trashv = lax.iota(jnp.int32, _LANES) + (TRASH0
                                                        + (cc % 2) * _LANES)
                idx_r[slot, sl] = w >> 14
                dst_r[slot, sl] = jnp.where(tok == 63, trashv, seg + base_dst)

        for b in range(_NB - 1):
            unpack_ring(b, b)
            pltpu.async_copy(emb_h.at[idx_r.at[b]], rows_v.at[b], gsems[b])

        def wait_g(sem):
            pltpu.make_async_copy(emb_h.at[idx_r.at[0]], rows_v.at[0],
                                  sem).wait()

        def wait_s(sem):
            pltpu.make_async_copy(rows_v.at[0], acc_s.at[pl.ds(0, _D)],
                                  sem).wait()

        def ring_fn(k, carry2):
            for p in range(_NB):
                r = _NB * k + p
                bn = (p + _NB - 1) % _NB
                wait_g(gsems[p])
                pltpu.async_copy(rows_v.at[p], acc_s.at[dst_r.at[p]],
                                 ssems[p], add=True)

                @pl.when(r + _NB - 1 < nr)
                def _():
                    @pl.when(r > 0)
                    def _():
                        wait_s(ssems[bn])
                    unpack_ring(r + _NB - 1, bn)
                    pltpu.async_copy(emb_h.at[idx_r.at[bn]],
                                     rows_v.at[bn], gsems[bn])
            return carry2

        lax.fori_loop(0, kk, ring_fn, 0)
        for b in range(_NB):
            wait_s(ssems[b])

        pltpu.sync_copy(acc_s.at[pl.ds(s * S, S)], out_h.at[pl.ds(t * S, S)])

    return bag(emb, sid3d, lens, zblk)


def _dense(sums, len1c, len2c, W1, b1, W2, b2, interpret=False):
    Bn2, D = sums.shape
    Bn = Bn2 // 2
    H = W1.shape[1]
    C = W2.shape[1]
    BLK = 1024
    nblk = Bn // BLK

    def body(us_r, vs_r, l1_r, l2_r, W1_r, b1_r, W2_r, b2_r, y_r, u_r, v_r):
        d1 = jnp.maximum(l1_r[...], 1).astype(jnp.float32)
        d2 = jnp.maximum(l2_r[...], 1).astype(jnp.float32)
        u = us_r[...] / d1
        v = vs_r[...] / d2
        u_r[...] = u
        v_r[...] = v
        a = jnp.abs(u - v)
        m = u * v
        h = (jnp.dot(u, W1_r[0:D, :], preferred_element_type=jnp.float32)
             + jnp.dot(v, W1_r[D:2 * D, :], preferred_element_type=jnp.float32)
             + jnp.dot(a, W1_r[2 * D:3 * D, :], preferred_element_type=jnp.float32)
             + jnp.dot(m, W1_r[3 * D:4 * D, :], preferred_element_type=jnp.float32)
             + b1_r[...])
        h = jnp.maximum(h, 0.0)
        y_r[...] = jnp.dot(h, W2_r[...], preferred_element_type=jnp.float32) + b2_r[...]

    return pl.pallas_call(
        body,
        grid=(nblk,),
        in_specs=[
            pl.BlockSpec((BLK, D), lambda i: (i, 0)),
            pl.BlockSpec((BLK, D), lambda i: (i + nblk, 0)),
            pl.BlockSpec((BLK, 1), lambda i: (i, 0)),
            pl.BlockSpec((BLK, 1), lambda i: (i, 0)),
            pl.BlockSpec((4 * D, H), lambda i: (0, 0)),
            pl.BlockSpec((1, H), lambda i: (0, 0)),
            pl.BlockSpec((H, C), lambda i: (0, 0)),
            pl.BlockSpec((1, C), lambda i: (0, 0)),
        ],
        out_specs=[
            pl.BlockSpec((BLK, C), lambda i: (i, 0)),
            pl.BlockSpec((BLK, D), lambda i: (i, 0)),
            pl.BlockSpec((BLK, D), lambda i: (i, 0)),
        ],
        out_shape=[
            jax.ShapeDtypeStruct((Bn, C), jnp.float32),
            jax.ShapeDtypeStruct((Bn, D), jnp.float32),
            jax.ShapeDtypeStruct((Bn, D), jnp.float32),
        ],
        interpret=interpret,
    )(sums, sums, len1c, len2c, W1, b1, W2, b2)


def kernel(sid1, sid2, len1, len2, emb, W1, b1, W2, b2, interpret=False):
    B, L = sid1.shape
    rounds = (2 * B // _NT) * _SL // _D
    sid3d = jnp.concatenate([sid1, sid2], axis=0).astype(jnp.int32)
    sid3d = sid3d.reshape(_NT, rounds, _D)
    lens = jnp.concatenate([len1, len2], axis=0).astype(jnp.int32)
    l1 = len1.astype(jnp.int32)
    l2 = len2.astype(jnp.int32)
    zblk = jnp.zeros((_D, _D), jnp.float32)

    sums = _embed_bag_sums(emb, sid3d, lens, zblk, 2 * B,
                           interpret=interpret)
    y, u, v = _dense(sums, l1[:, None], l2[:, None], W1, b1[None, :],
                     W2, b2[None, :], interpret=interpret)
    return (y, u, v)
